# baseline (speedup 1.0000x reference)
import jax
import jax.numpy as jnp
from jax.experimental import pallas as pl
from jax.experimental.pallas import tpu as pltpu

NV_PREV = 10242
BLOCK = 1138


def _copy_body(x_ref, o_ref):
    o_ref[...] = x_ref[...]


def kernel(x):
    n, a, b, c = x.shape
    x2 = x.reshape(n, 8, 128)
    out2 = pl.pallas_call(
        _copy_body,
        grid=(NV_PREV // BLOCK,),
        in_specs=[pl.BlockSpec((BLOCK, 8, 128), lambda i: (i, 0, 0))],
        out_specs=pl.BlockSpec((BLOCK, 8, 128), lambda i: (i, 0, 0)),
        out_shape=jax.ShapeDtypeStruct((NV_PREV, 8, 128), x.dtype),
        compiler_params=pltpu.CompilerParams(
            dimension_semantics=("parallel",)),
    )(x2)
    return out2.reshape(NV_PREV, a, b, c)


# speedup vs baseline: 1.0037x; 1.0037x over previous
import jax
import jax.numpy as jnp
from jax.experimental import pallas as pl
from jax.experimental.pallas import tpu as pltpu

NV_PREV = 10242
CHUNKS = 18
CH = NV_PREV // CHUNKS
assert CH * CHUNKS == NV_PREV


def _copy_body(x_ref, o_ref, buf, in_sems, out_sems):
    in_cps = []
    for g in range(CHUNKS):
        cp = pltpu.make_async_copy(
            x_ref.at[pl.ds(g * CH, CH)], buf.at[g], in_sems.at[g])
        cp.start()
        in_cps.append(cp)
    out_cps = []
    for g in range(CHUNKS):
        in_cps[g].wait()
        cp = pltpu.make_async_copy(
            buf.at[g], o_ref.at[pl.ds(g * CH, CH)], out_sems.at[g])
        cp.start()
        out_cps.append(cp)
    for cp in out_cps:
        cp.wait()


def kernel(x):
    n, a, b, c = x.shape
    x2 = x.reshape(n, 8, 128)
    out2 = pl.pallas_call(
        _copy_body,
        out_shape=jax.ShapeDtypeStruct((NV_PREV, 8, 128), x.dtype),
        in_specs=[pl.BlockSpec(memory_space=pl.ANY)],
        out_specs=pl.BlockSpec(memory_space=pl.ANY),
        scratch_shapes=[
            pltpu.VMEM((CHUNKS, CH, 8, 128), x.dtype),
            pltpu.SemaphoreType.DMA((CHUNKS,)),
            pltpu.SemaphoreType.DMA((CHUNKS,)),
        ],
    )(x2)
    return out2.reshape(NV_PREV, a, b, c)
